# Initial kernel scaffold; baseline (speedup 1.0000x reference)
import jax
import jax.numpy as jnp
from jax.experimental import pallas as pl


def kernel(traj, mat, traj_len, emb_t, emb_l, emb_u, emb_su, emb_sl, emb_tu, emb_tl):
    raise NotImplementedError("write your pallas kernel here")


# speedup vs baseline: 10.1446x; 10.1446x over previous
import functools

import jax
import jax.numpy as jnp
from jax import lax
from jax.experimental import pallas as pl
from jax.experimental.pallas import tpu as pltpu
from jax.experimental.pallas import tpu_sc as plsc

HOURS = 168
SU, SL, TU, TL = 100.0, 0.0, 3600.0, 0.0
EMB = 32
N, M = 1024, 20

NC, NS = 2, 16
NW = NC * NS
ROWS = N * M
BPW = ROWS // NW

BB = 128
LANES = 128


def _joint_body(idx_t, idx_l, idx_u, emb_t, emb_l, emb_u, out,
                it_v, il_v, iu_v, rt, rl, ru, sem_t, sem_l, sem_u):
    wid = lax.axis_index("s") * NC + lax.axis_index("c")
    base = wid * BPW
    pltpu.sync_copy(idx_t.at[pl.ds(base, BPW)], it_v)
    pltpu.sync_copy(idx_l.at[pl.ds(base, BPW)], il_v)
    pltpu.sync_copy(idx_u.at[pl.ds(base, BPW)], iu_v)
    ct = pltpu.async_copy(emb_t.at[it_v], rt, sem_t)
    cl = pltpu.async_copy(emb_l.at[il_v], rl, sem_l)
    cu = pltpu.async_copy(emb_u.at[iu_v], ru, sem_u)
    ct.wait()
    cl.wait()
    cu.wait()

    def row(r, carry):
        rl[r, pl.ds(0, 16)] = rt[r, pl.ds(0, 16)] + rl[r, pl.ds(0, 16)] + ru[r, pl.ds(0, 16)]
        rl[r, pl.ds(16, 16)] = rt[r, pl.ds(16, 16)] + rl[r, pl.ds(16, 16)] + ru[r, pl.ds(16, 16)]
        return carry

    lax.fori_loop(0, BPW, row, 0)
    pltpu.sync_copy(rl, out.at[pl.ds(base, BPW)])


@functools.lru_cache(maxsize=1)
def _joint_call():
    return pl.kernel(
        _joint_body,
        out_type=jax.ShapeDtypeStruct((ROWS, EMB), jnp.float32),
        mesh=plsc.VectorSubcoreMesh(core_axis_name="c", subcore_axis_name="s",
                                    num_cores=NC, num_subcores=NS),
        scratch_types=[
            pltpu.VMEM((BPW,), jnp.int32),
            pltpu.VMEM((BPW,), jnp.int32),
            pltpu.VMEM((BPW,), jnp.int32),
            pltpu.VMEM((BPW, EMB), jnp.float32),
            pltpu.VMEM((BPW, EMB), jnp.float32),
            pltpu.VMEM((BPW, EMB), jnp.float32),
            pltpu.SemaphoreType.DMA,
            pltpu.SemaphoreType.DMA,
            pltpu.SemaphoreType.DMA,
        ],
        compiler_params=pltpu.CompilerParams(use_tc_tiling_on_sc=False),
    )


def _tile4(row):
    return jnp.concatenate([row] * 4, axis=1)[:, None, :]


def _delta_body(len_ref, mat_ref, esl, esu, etl, etu, out_ref):
    A = esl[...] + etl[...]
    S = (esu[...] - esl[...]) * (1.0 / (SU - SL))
    T = (etu[...] - etl[...]) * (1.0 / (TU - TL))
    A0, A1 = _tile4(A[0:1, :]), _tile4(A[1:2, :])
    S0, S1 = _tile4(S[0:1, :]), _tile4(S[1:2, :])
    T0, T1 = _tile4(T[0:1, :]), _tile4(T[1:2, :])

    len3 = len_ref[...][:, :, None]
    vi = lax.broadcasted_iota(jnp.int32, (BB, M, LANES), 1) < len3
    jl = lax.broadcasted_iota(jnp.int32, (BB, M, LANES), 2) // EMB

    for g in range(M * EMB // LANES):
        m = vi & ((jl + 4 * g) < len3)
        ds_parts, dt_parts = [], []
        for k in range(4):
            j = 4 * g + k
            ds_parts.append(jnp.broadcast_to(mat_ref[:, :, 2 * j:2 * j + 1], (BB, M, EMB)))
            dt_parts.append(jnp.broadcast_to(mat_ref[:, :, 2 * j + 1:2 * j + 2], (BB, M, EMB)))
        dsg = jnp.concatenate(ds_parts, axis=2)
        dtg = jnp.concatenate(dt_parts, axis=2)
        Am = jnp.where(m, A1, A0)
        Sm = jnp.where(m, S1, S0)
        Tm = jnp.where(m, T1, T0)
        out_ref[:, :, LANES * g:LANES * (g + 1)] = Am + dsg * Sm + dtg * Tm


_delta_call = pl.pallas_call(
    _delta_body,
    grid=(N // BB,),
    in_specs=[
        pl.BlockSpec((BB, 1), lambda i: (i, 0)),
        pl.BlockSpec((BB, M, 2 * M), lambda i: (i, 0, 0)),
        pl.BlockSpec((2, EMB), lambda i: (0, 0)),
        pl.BlockSpec((2, EMB), lambda i: (0, 0)),
        pl.BlockSpec((2, EMB), lambda i: (0, 0)),
        pl.BlockSpec((2, EMB), lambda i: (0, 0)),
    ],
    out_specs=pl.BlockSpec((BB, M, M * EMB), lambda i: (i, 0, 0)),
    out_shape=jax.ShapeDtypeStruct((N, M, M * EMB), jnp.float32),
)


def kernel(traj, mat, traj_len, emb_t, emb_l, emb_u, emb_su, emb_sl, emb_tu, emb_tl):
    traj = traj.astype(jnp.int32)
    idx_u = traj[:, :, 0].reshape(ROWS)
    idx_l = traj[:, :, 1].reshape(ROWS)
    idx_t = ((traj[:, :, 2] - 1) % HOURS + 1).reshape(ROWS)
    joint = _joint_call()(idx_t, idx_l, idx_u, emb_t, emb_l, emb_u)
    len2d = traj_len.astype(jnp.int32).reshape(N, 1)
    mat_r = mat.reshape(N, M, 2 * M)
    delta = _delta_call(len2d, mat_r, emb_sl, emb_su, emb_tl, emb_tu)
    return joint.reshape(N, M, EMB), delta.reshape(N, M, M, EMB)
